# baseline (speedup 1.0000x reference)
import functools

import jax
import jax.numpy as jnp
from jax import lax
from jax.experimental import pallas as pl
from jax.experimental.pallas import tpu as pltpu
from jax.experimental.pallas import tpu_sc as plsc

_GRID = 128
_NSH = 9
_N = 1048576

_NC = 2
_NS = 16
_NW = _NC * _NS
_PW = _N // _NW
_C = 256
_G = _PW // _C
_TOTAL_CHUNKS = _N // _C
_SLICE = 128
_NSLICE = (_C * 8) // _SLICE

_SH0 = 0.28209479177387814
_SH1 = 0.4886025119029199
_SH4 = 1.0925484305920792
_SH6 = 0.31539156525252005
_SH8 = 0.5462742152960396

_CORNER_OFF = [dx * _GRID * _GRID + dy * _GRID + dz
               for dx in (0, 1) for dy in (0, 1) for dz in (0, 1)]


def _sc_body(table, pts, nrm, par, out, par_v, pts_v, nrm_v, out_v,
             aux0, aux1, idx0, idx1, rows0, rows1, sem0, sem1):
    wid = lax.axis_index("s") * _NC + lax.axis_index("c")
    pltpu.sync_copy(par, par_v)

    iota = lax.iota(jnp.int32, 16)
    kvecs = [jnp.full((16,), k, jnp.int32) for k in range(_NSH)]
    scale = [par_v[d, :] for d in range(3)]
    shift = [par_v[3 + d, :] for d in range(3)]

    def load_in(g):
        base = wid * _PW + g * _C
        pltpu.sync_copy(pts.at[pl.ds(base, _C)], pts_v)
        pltpu.sync_copy(nrm.at[pl.ds(base, _C)], nrm_v)

    def pass1(aux, idx):
        def grp(i, carry):
            s16 = pl.ds(i * 16, 16)
            rowv = iota + i * 16
            p = [plsc.load_gather(pts_v, [rowv, kvecs[d]]) for d in range(3)]
            nx, ny, nz = (plsc.load_gather(nrm_v, [rowv, kvecs[d]])
                          for d in range(3))
            ii = []
            for d in range(3):
                pos = jnp.clip(p[d] * scale[d] + shift[d], 0.0, float(_GRID - 1))
                i0 = jnp.minimum(pos.astype(jnp.int32), _GRID - 2)
                aux[d, s16] = pos - i0.astype(jnp.float32)
                ii.append(i0)
            base = (ii[0] * _GRID + ii[1]) * _GRID + ii[2]
            aux[3, s16] = -_SH1 * ny
            aux[4, s16] = _SH1 * nz
            aux[5, s16] = -_SH1 * nx
            aux[6, s16] = _SH4 * (nx * ny)
            aux[7, s16] = -_SH4 * (ny * nz)
            aux[8, s16] = _SH6 * (2.0 * nz * nz - nx * nx - ny * ny)
            aux[9, s16] = -_SH4 * (nx * nz)
            aux[10, s16] = _SH8 * (nx * nx - ny * ny)
            ibase = i * 16
            for c in range(8):
                idx[pl.ds(c * _C + ibase, 16)] = base + _CORNER_OFF[c]
            return carry
        lax.fori_loop(0, _C // 16, grp, 0)

    def fire(idx, rows, sem):
        for j in range(_NSLICE):
            pltpu.async_copy(table.at[idx.at[pl.ds(j * _SLICE, _SLICE)]],
                             rows.at[pl.ds(j * _SLICE, _SLICE)], sem)

    def drain(idx, rows, sem):
        for j in range(_NSLICE):
            pltpu.make_async_copy(table.at[idx.at[pl.ds(j * _SLICE, _SLICE)]],
                                  rows.at[pl.ds(j * _SLICE, _SLICE)], sem).wait()

    def pass2(aux, rows, g):
        def grp(i, carry):
            s16 = pl.ds(i * 16, 16)
            fx, fy, fz = aux[0, s16], aux[1, s16], aux[2, s16]
            wx = [1.0 - fx, fx]
            wy = [1.0 - fy, fy]
            wz = [1.0 - fz, fz]
            sh = [None] + [aux[3 + j, s16] for j in range(8)]
            ibase = i * 16
            acc = None
            for c in range(8):
                dx, dy, dz = c >> 2, (c >> 1) & 1, c & 1
                d0 = iota + (c * _C + ibase)
                dot = _SH0 * plsc.load_gather(rows, [d0, kvecs[0]])
                for k in range(1, _NSH):
                    dot = dot + sh[k] * plsc.load_gather(rows, [d0, kvecs[k]])
                w = wx[dx] * wy[dy] * wz[dz]
                acc = w * dot if acc is None else acc + w * dot
            out_v[s16] = jnp.maximum(acc, 0.0)
            return carry
        lax.fori_loop(0, _C // 16, grp, 0)
        pltpu.sync_copy(out_v, out.at[pl.ds(wid * _PW + g * _C, _C)])

    load_in(0)
    pass1(aux0, idx0)
    fire(idx0, rows0, sem0)

    def outer(t, carry):
        g = t * 2
        load_in(g + 1)
        pass1(aux1, idx1)
        fire(idx1, rows1, sem1)
        drain(idx0, rows0, sem0)
        pass2(aux0, rows0, g)

        @pl.when(g + 2 < _G)
        def _():
            load_in(g + 2)
            pass1(aux0, idx0)
            fire(idx0, rows0, sem0)
        drain(idx1, rows1, sem1)
        pass2(aux1, rows1, g + 1)
        return carry

    lax.fori_loop(0, _G // 2, outer, 0)


@functools.partial(jax.jit, static_argnums=())
def _sc_call(table, pts, nrm, par):
    mesh = plsc.VectorSubcoreMesh(core_axis_name="c", subcore_axis_name="s",
                                  num_cores=_NC, num_subcores=_NS)
    f = pl.kernel(
        _sc_body,
        out_type=jax.ShapeDtypeStruct((_N,), jnp.float32),
        mesh=mesh,
        compiler_params=pltpu.CompilerParams(needs_layout_passes=False,
                                             use_tc_tiling_on_sc=False),
        scratch_types=[
            pltpu.VMEM((6, 16), jnp.float32),
            pltpu.VMEM((_C, 3), jnp.float32),
            pltpu.VMEM((_C, 3), jnp.float32),
            pltpu.VMEM((_C,), jnp.float32),
            pltpu.VMEM((11, _C), jnp.float32),
            pltpu.VMEM((11, _C), jnp.float32),
            pltpu.VMEM((_C * 8,), jnp.int32),
            pltpu.VMEM((_C * 8,), jnp.int32),
            pltpu.VMEM((_C * 8, _NSH), jnp.float32),
            pltpu.VMEM((_C * 8, _NSH), jnp.float32),
            pltpu.SemaphoreType.DMA,
            pltpu.SemaphoreType.DMA,
        ],
    )
    return f(table, pts, nrm, par)


def kernel(points, normals, coefficients, aabb):
    table = coefficients.reshape(_GRID * _GRID * _GRID, _NSH)
    scale = (_GRID - 1) / (aabb[3:] - aabb[:3])
    shift = -aabb[:3] * scale
    par = jnp.broadcast_to(jnp.concatenate([scale, shift])[:, None], (6, 16))
    par = par.astype(jnp.float32)
    out = _sc_call(table, points, normals, par)
    return out.reshape(_N, 1)


# speedup vs baseline: 2.3562x; 2.3562x over previous
import functools

import jax
import jax.numpy as jnp
from jax import lax
from jax.experimental import pallas as pl
from jax.experimental.pallas import tpu as pltpu
from jax.experimental.pallas import tpu_sc as plsc

_GRID = 128
_NSH = 9
_N = 1048576

_NC = 2
_NS = 16
_NW = _NC * _NS
_PW = _N // _NW
_C = 256
_G = _PW // _C
_TOTAL_CHUNKS = _N // _C
_SLICE = 128
_NSLICE = (_C * 8) // _SLICE

_SH0 = 0.28209479177387814
_SH1 = 0.4886025119029199
_SH4 = 1.0925484305920792
_SH6 = 0.31539156525252005
_SH8 = 0.5462742152960396

_CORNER_OFF = [dx * _GRID * _GRID + dy * _GRID + dz
               for dx in (0, 1) for dy in (0, 1) for dz in (0, 1)]


def _sc_body(table, inp, par, out, par_v, in_v, out_v,
             aux0, aux1, idx0, idx1, rows0, rows1, sem0, sem1):
    wid = lax.axis_index("s") * _NC + lax.axis_index("c")
    pltpu.sync_copy(par, par_v)

    iota = lax.iota(jnp.int32, 16)
    kvecs = [jnp.full((16,), k, jnp.int32) for k in range(_NSH)]
    scale = [par_v[pl.ds(d * 16, 16)] for d in range(3)]
    shift = [par_v[pl.ds((3 + d) * 16, 16)] for d in range(3)]

    def load_in(g):
        pltpu.sync_copy(inp.at[pl.ds((wid * _G + g) * 6 * _C, 6 * _C)], in_v)

    def pass1(aux, idx):
        def grp(i, carry):
            s16 = pl.ds(i * 16, 16)
            p = [in_v[pl.ds(d * _C + i * 16, 16)] for d in range(3)]
            nx, ny, nz = (in_v[pl.ds((3 + d) * _C + i * 16, 16)]
                          for d in range(3))
            ii = []
            for d in range(3):
                pos = jnp.clip(p[d] * scale[d] + shift[d], 0.0, float(_GRID - 1))
                i0 = jnp.minimum(pos.astype(jnp.int32), _GRID - 2)
                aux[d, s16] = pos - i0.astype(jnp.float32)
                ii.append(i0)
            base = (ii[0] * _GRID + ii[1]) * _GRID + ii[2]
            aux[3, s16] = -_SH1 * ny
            aux[4, s16] = _SH1 * nz
            aux[5, s16] = -_SH1 * nx
            aux[6, s16] = _SH4 * (nx * ny)
            aux[7, s16] = -_SH4 * (ny * nz)
            aux[8, s16] = _SH6 * (2.0 * nz * nz - nx * nx - ny * ny)
            aux[9, s16] = -_SH4 * (nx * nz)
            aux[10, s16] = _SH8 * (nx * nx - ny * ny)
            ibase = i * 16
            for c in range(8):
                idx[pl.ds(c * _C + ibase, 16)] = base + _CORNER_OFF[c]
            return carry
        lax.fori_loop(0, _C // 16, grp, 0)

    def fire(idx, rows, sem):
        for j in range(_NSLICE):
            pltpu.async_copy(table.at[idx.at[pl.ds(j * _SLICE, _SLICE)]],
                             rows.at[pl.ds(j * _SLICE, _SLICE)], sem)

    def drain(idx, rows, sem):
        for j in range(_NSLICE):
            pltpu.make_async_copy(table.at[idx.at[pl.ds(j * _SLICE, _SLICE)]],
                                  rows.at[pl.ds(j * _SLICE, _SLICE)], sem).wait()

    def pass2(aux, rows, g):
        def grp(i, carry):
            s16 = pl.ds(i * 16, 16)
            fx, fy, fz = aux[0, s16], aux[1, s16], aux[2, s16]
            wx = [1.0 - fx, fx]
            wy = [1.0 - fy, fy]
            wz = [1.0 - fz, fz]
            sh = [None] + [aux[3 + j, s16] for j in range(8)]
            ibase = i * 16
            acc = None
            for c in range(8):
                dx, dy, dz = c >> 2, (c >> 1) & 1, c & 1
                d0 = iota + (c * _C + ibase)
                dot = _SH0 * plsc.load_gather(rows, [d0, kvecs[0]])
                for k in range(1, _NSH):
                    dot = dot + sh[k] * plsc.load_gather(rows, [d0, kvecs[k]])
                w = wx[dx] * wy[dy] * wz[dz]
                acc = w * dot if acc is None else acc + w * dot
            out_v[s16] = jnp.maximum(acc, 0.0)
            return carry
        lax.fori_loop(0, _C // 16, grp, 0)
        pltpu.sync_copy(out_v, out.at[pl.ds(wid * _PW + g * _C, _C)])

    load_in(0)
    pass1(aux0, idx0)
    fire(idx0, rows0, sem0)

    def outer(t, carry):
        g = t * 2
        load_in(g + 1)
        pass1(aux1, idx1)
        fire(idx1, rows1, sem1)
        drain(idx0, rows0, sem0)
        pass2(aux0, rows0, g)

        @pl.when(g + 2 < _G)
        def _():
            load_in(g + 2)
            pass1(aux0, idx0)
            fire(idx0, rows0, sem0)
        drain(idx1, rows1, sem1)
        pass2(aux1, rows1, g + 1)
        return carry

    lax.fori_loop(0, _G // 2, outer, 0)


@functools.partial(jax.jit, static_argnums=())
def _sc_call(table, inp, par):
    mesh = plsc.VectorSubcoreMesh(core_axis_name="c", subcore_axis_name="s",
                                  num_cores=_NC, num_subcores=_NS)
    f = pl.kernel(
        _sc_body,
        out_type=jax.ShapeDtypeStruct((_N,), jnp.float32),
        mesh=mesh,
        compiler_params=pltpu.CompilerParams(needs_layout_passes=False,
                                             use_tc_tiling_on_sc=False),
        scratch_types=[
            pltpu.VMEM((96,), jnp.float32),
            pltpu.VMEM((6 * _C,), jnp.float32),
            pltpu.VMEM((_C,), jnp.float32),
            pltpu.VMEM((11, _C), jnp.float32),
            pltpu.VMEM((11, _C), jnp.float32),
            pltpu.VMEM((_C * 8,), jnp.int32),
            pltpu.VMEM((_C * 8,), jnp.int32),
            pltpu.VMEM((_C * 8, _NSH), jnp.float32),
            pltpu.VMEM((_C * 8, _NSH), jnp.float32),
            pltpu.SemaphoreType.DMA,
            pltpu.SemaphoreType.DMA,
        ],
    )
    return f(table, inp, par)


def kernel(points, normals, coefficients, aabb):
    table = coefficients.reshape(_GRID * _GRID * _GRID, _NSH)
    comp = jnp.concatenate([points.T, normals.T], axis=0)
    inp = comp.reshape(6, _TOTAL_CHUNKS, _C).transpose(1, 0, 2).reshape(-1)
    scale = (_GRID - 1) / (aabb[3:] - aabb[:3])
    shift = -aabb[:3] * scale
    par = jnp.broadcast_to(jnp.concatenate([scale, shift])[:, None], (6, 16))
    par = par.astype(jnp.float32).reshape(-1)
    out = _sc_call(table, inp, par)
    return out.reshape(_N, 1)
